# baseline (speedup 1.0000x reference)
import functools
import math

import jax
import jax.numpy as jnp
from jax import lax
from jax.experimental import pallas as pl
from jax.experimental.pallas import tpu as pltpu, tpu_sc as plsc

MU_CONST = 3.5
HLEN = 200
D = 32
L = 16
NA = 104
NB = HLEN - NA
BIAS_ROW = 16


def _sc_body(uidx_hbm, iidx_hbm, iu_hbm, uemb_hbm, iemb_hbm, ub_hbm, ib_hbm,
             yj_hbm, w_hbm, b_hbm, out_hbm,
             uq_v, iq_v, iua_v, iub_v, rows_a, rows_b,
             urow_v, irow_v, ubrow_v, ibrow_v, w_v, b_v, ur_v, ir_v, res_v,
             sem, gsem):
  is_w0 = jnp.logical_and(lax.axis_index("c") == 0, lax.axis_index("s") == 0)

  @pl.when(is_w0)
  def _():
    iota = lax.iota(jnp.int32, L)
    lane0 = iota == 0
    zf = jnp.zeros((L,), jnp.float32)
    zi = jnp.zeros((L,), jnp.int32)

    c1 = pltpu.async_copy(uidx_hbm, uq_v.at[pl.ds(0, 1)], sem)
    c2 = pltpu.async_copy(iidx_hbm, iq_v.at[pl.ds(0, 1)], sem)
    c3 = pltpu.async_copy(iu_hbm.at[pl.ds(0, NA)], iua_v, sem)
    c4 = pltpu.async_copy(iu_hbm.at[pl.ds(NA, NB)], iub_v, sem)
    c5 = pltpu.async_copy(w_hbm, w_v, sem)
    c6 = pltpu.async_copy(b_hbm, b_v.at[pl.ds(0, 1)], sem)
    c1.wait(); c2.wait(); c3.wait(); c4.wait(); c5.wait(); c6.wait()

    g1 = pltpu.async_copy(yj_hbm.at[iua_v], rows_a, gsem)
    g2 = pltpu.async_copy(yj_hbm.at[iub_v], rows_b, gsem)
    g3 = pltpu.async_copy(uemb_hbm.at[uq_v.at[pl.ds(0, 1)]], urow_v, gsem)
    g4 = pltpu.async_copy(iemb_hbm.at[iq_v.at[pl.ds(0, 1)]], irow_v, gsem)

    uvec = uq_v[...]
    ivec = iq_v[...]
    ur_v[...] = lax.shift_right_logical(uvec, 4)
    ir_v[...] = lax.shift_right_logical(ivec, 4)
    g5 = pltpu.async_copy(ub_hbm.at[ur_v.at[pl.ds(0, 1)]], ubrow_v, gsem)
    g6 = pltpu.async_copy(ib_hbm.at[ir_v.at[pl.ds(0, 1)]], ibrow_v, gsem)
    g1.wait(); g2.wait(); g3.wait(); g4.wait(); g5.wait(); g6.wait()

    def acc_a(k, carry):
      a0, a1 = carry
      return a0 + rows_a[k, pl.ds(0, L)], a1 + rows_a[k, pl.ds(L, L)]

    def acc_b(k, carry):
      a0, a1 = carry
      return a0 + rows_b[k, pl.ds(0, L)], a1 + rows_b[k, pl.ds(L, L)]

    s0, s1 = lax.fori_loop(0, NA, acc_a, (zf, zf))
    s0, s1 = lax.fori_loop(0, NB, acc_b, (s0, s1))

    inv = jnp.float32(1.0 / math.sqrt(HLEN))
    uv0 = urow_v[0, pl.ds(0, L)] + s0 * inv
    uv1 = urow_v[0, pl.ds(L, L)] + s1 * inv
    p = uv0 * irow_v[0, pl.ds(0, L)] * w_v[0, pl.ds(0, L)] \
        + uv1 * irow_v[0, pl.ds(L, L)] * w_v[0, pl.ds(L, L)]
    dot = jnp.sum(p)

    ulane = jnp.sum(jnp.where(lane0, jnp.bitwise_and(uvec, 15), zi))
    ilane = jnp.sum(jnp.where(lane0, jnp.bitwise_and(ivec, 15), zi))
    ubias = jnp.sum(jnp.where(iota == ulane, ubrow_v[0, pl.ds(0, L)], zf))
    ibias = jnp.sum(jnp.where(iota == ilane, ibrow_v[0, pl.ds(0, L)], zf))
    bval = jnp.sum(jnp.where(lane0, b_v[...], zf))

    rating = dot + bval + jnp.float32(MU_CONST) + ubias + ibias
    res_v[...] = jnp.full((L,), rating, jnp.float32)
    pltpu.sync_copy(res_v, out_hbm)


@functools.partial(
    pl.kernel,
    out_type=jax.ShapeDtypeStruct((L,), jnp.float32),
    mesh=plsc.VectorSubcoreMesh(core_axis_name="c", subcore_axis_name="s"),
    compiler_params=pltpu.CompilerParams(use_tc_tiling_on_sc=False,
                                         needs_layout_passes=False),
    scratch_types=[
        pltpu.VMEM((L,), jnp.int32),
        pltpu.VMEM((L,), jnp.int32),
        pltpu.VMEM((NA,), jnp.int32),
        pltpu.VMEM((NB,), jnp.int32),
        pltpu.VMEM((NA, D), jnp.float32),
        pltpu.VMEM((NB, D), jnp.float32),
        pltpu.VMEM((1, D), jnp.float32),
        pltpu.VMEM((1, D), jnp.float32),
        pltpu.VMEM((1, BIAS_ROW), jnp.float32),
        pltpu.VMEM((1, BIAS_ROW), jnp.float32),
        pltpu.VMEM((1, D), jnp.float32),
        pltpu.VMEM((L,), jnp.float32),
        pltpu.VMEM((L,), jnp.int32),
        pltpu.VMEM((L,), jnp.int32),
        pltpu.VMEM((L,), jnp.float32),
        pltpu.SemaphoreType.DMA,
        pltpu.SemaphoreType.DMA,
    ],
)
def _svdpp_sc(*refs):
  _sc_body(*refs)


def kernel(user_idx, item_idx, Iu, user_embedding, item_embedding, user_bias,
           item_bias, yj, W, b):
  ub2 = user_bias.reshape(-1, BIAS_ROW)
  ib2 = item_bias.reshape(-1, BIAS_ROW)
  out = _svdpp_sc(user_idx, item_idx, Iu, user_embedding, item_embedding,
                  ub2, ib2, yj, W, b)
  return out[:1].reshape(1, 1)


# speedup vs baseline: 45.7299x; 45.7299x over previous
import functools
import math

import jax
import jax.numpy as jnp
from jax import lax
from jax.experimental import pallas as pl
from jax.experimental.pallas import tpu as pltpu, tpu_sc as plsc

MU_CONST = 3.5
HLEN = 200
D = 32
L = 16
N_TABLE = 1000000
TAIL_START = (N_TABLE // 128) * 128
LAST_BLOCK = TAIL_START - 128
N_TAIL = N_TABLE - TAIL_START
N_YJW = 13


def _col_lookup(tblT_hbm, tail_ref, blk_slot, idx, iota, sem):
  blkoff = lax.shift_left(lax.shift_right_logical(idx, 7), 7)
  safe = pl.multiple_of(jnp.minimum(blkoff, LAST_BLOCK), 128)
  h = pltpu.async_copy(tblT_hbm.at[:, pl.ds(safe, 128)], blk_slot, sem)
  col = jnp.minimum(idx - safe, 127)
  trow = jnp.maximum(idx - TAIL_START, 0)
  return h, col, trow


def _extract(blk_ref, slot, tail_ref, col, trow, idx, iota):
  colv = jnp.full((L,), col, jnp.int32)
  slotv = jnp.full((L,), slot, jnp.int32)
  tbase = jnp.full((L,), trow * D, jnp.int32) + iota
  lo = plsc.load_gather(blk_ref, [slotv, iota, colv])
  hi = plsc.load_gather(blk_ref, [slotv, iota + 16, colv])
  tlo = plsc.load_gather(tail_ref, [tbase])
  thi = plsc.load_gather(tail_ref, [tbase + 16])
  is_tail = idx >= TAIL_START
  return jnp.where(is_tail, tlo, lo), jnp.where(is_tail, thi, hi)


def _sc_body(uidx_hbm, iidx_hbm, iu_hbm, uembT_hbm, iembT_hbm, ub_hbm, ib_hbm,
             yjT_hbm, w_hbm, b_hbm, utail_hbm, itail_hbm, ytail_hbm, out_hbm,
             iu_v, q_v, blk_v, tail_v, bias_v, part_v, w_v, b_v,
             shared, gath_v, res_v, sem, sem_idx, sem_tail, sem_bias):
  cid = lax.axis_index("c")
  sid = lax.axis_index("s")

  @pl.when(cid == 0)
  def _():
    iota = lax.iota(jnp.int32, L)
    zf = jnp.zeros((L,), jnp.float32)

    @pl.when(sid < N_YJW)
    def _():
      base = pl.multiple_of(sid * L, 16)
      c0 = pltpu.async_copy(iu_hbm, iu_v, sem_idx)
      c0.wait()
      ct = pltpu.async_copy(ytail_hbm, tail_v, sem_tail)
      vec = plsc.load_gather(iu_v, [jnp.full((L,), base, jnp.int32) + iota])
      handles, cols, trows, idxs = [], [], [], []
      for k in range(L):
        i_k = vec[k]
        h, col, trow = _col_lookup(yjT_hbm, tail_v, blk_v.at[k], i_k, iota, sem)
        handles.append(h)
        cols.append(col)
        trows.append(trow)
        idxs.append(i_k)
      ct.wait()
      for h in handles:
        h.wait()
      acc0, acc1 = zf, zf
      for k in range(L):
        lo, hi = _extract(blk_v, k, tail_v, cols[k], trows[k], idxs[k], iota)
        valid = (base + k) < HLEN
        acc0 = acc0 + jnp.where(valid, lo, zf)
        acc1 = acc1 + jnp.where(valid, hi, zf)
      part_v[0, pl.ds(L, L)] = acc0
      part_v[0, pl.ds(2 * L, L)] = acc1
      pltpu.sync_copy(part_v, shared.at[pl.ds(sid, 1)])

    def row_work(idx_hbm, tblT_hbm, tail_hbm, bias_hbm, row_slot, bias_slot):
      c0 = pltpu.async_copy(idx_hbm, q_v.at[pl.ds(0, 1)], sem_idx)
      c0.wait()
      ct = pltpu.async_copy(tail_hbm, tail_v, sem_tail)
      u = q_v[...][0]
      h, col, trow = _col_lookup(tblT_hbm, tail_v, blk_v.at[0], u, iota, sem)
      boff = pl.multiple_of(lax.shift_left(lax.shift_right_logical(u, 4), 4), 16)
      hb = pltpu.async_copy(bias_hbm.at[pl.ds(boff, L)], bias_v, sem_bias)
      ct.wait(); h.wait(); hb.wait()
      lo, hi = _extract(blk_v, 0, tail_v, col, trow, u, iota)
      part_v[0, pl.ds(L, L)] = lo
      part_v[0, pl.ds(2 * L, L)] = hi
      pltpu.sync_copy(part_v, shared.at[pl.ds(row_slot, 1)])
      lane = jnp.bitwise_and(u, 15)
      sel = jnp.where(iota == lane, bias_v[...], zf)
      part_v[0, pl.ds(L, L)] = sel
      part_v[0, pl.ds(2 * L, L)] = zf
      pltpu.sync_copy(part_v, shared.at[pl.ds(bias_slot, 1)])

    @pl.when(sid == 13)
    def _():
      row_work(uidx_hbm, uembT_hbm, utail_hbm, ub_hbm, 13, 15)

    @pl.when(sid == 14)
    def _():
      row_work(iidx_hbm, iembT_hbm, itail_hbm, ib_hbm, 14, 16)

    plsc.subcore_barrier()

    @pl.when(sid == 0)
    def _():
      cw = pltpu.async_copy(w_hbm, w_v, sem_tail)
      cb = pltpu.async_copy(b_hbm, b_v.at[pl.ds(0, 1)], sem_bias)
      pltpu.sync_copy(shared, gath_v)
      s0, s1 = zf, zf
      for r in range(N_YJW):
        s0 = s0 + gath_v[r, pl.ds(L, L)]
        s1 = s1 + gath_v[r, pl.ds(2 * L, L)]
      cw.wait(); cb.wait()
      inv = jnp.float32(1.0 / math.sqrt(HLEN))
      uv0 = gath_v[13, pl.ds(L, L)] + s0 * inv
      uv1 = gath_v[13, pl.ds(2 * L, L)] + s1 * inv
      p = uv0 * gath_v[14, pl.ds(L, L)] * w_v[0, pl.ds(0, L)] \
          + uv1 * gath_v[14, pl.ds(2 * L, L)] * w_v[0, pl.ds(L, L)]
      dot = jnp.sum(p)
      ubias = jnp.sum(gath_v[15, pl.ds(L, L)])
      ibias = jnp.sum(gath_v[16, pl.ds(L, L)])
      bval = b_v[...][0]
      rating = dot + bval + jnp.float32(MU_CONST) + ubias + ibias
      res_v[...] = jnp.full((L,), rating, jnp.float32)
      pltpu.sync_copy(res_v, out_hbm)


@functools.partial(
    pl.kernel,
    out_type=jax.ShapeDtypeStruct((L,), jnp.float32),
    mesh=plsc.VectorSubcoreMesh(core_axis_name="c", subcore_axis_name="s"),
    compiler_params=pltpu.CompilerParams(use_tc_tiling_on_sc=True,
                                         needs_layout_passes=False),
    scratch_types=[
        pltpu.VMEM((HLEN + 8,), jnp.int32),
        pltpu.VMEM((L,), jnp.int32),
        pltpu.VMEM((L, D, 128), jnp.float32),
        pltpu.VMEM((N_TAIL * D,), jnp.float32),
        pltpu.VMEM((L,), jnp.float32),
        pltpu.VMEM((1, 2 * D), jnp.float32),
        pltpu.VMEM((1, D), jnp.float32),
        pltpu.VMEM((L,), jnp.float32),
        pltpu.VMEM_SHARED((17, 2 * D), jnp.float32),
        pltpu.VMEM((17, 2 * D), jnp.float32),
        pltpu.VMEM((L,), jnp.float32),
        pltpu.SemaphoreType.DMA,
        pltpu.SemaphoreType.DMA,
        pltpu.SemaphoreType.DMA,
        pltpu.SemaphoreType.DMA,
    ],
)
def _svdpp_sc(*refs):
  _sc_body(*refs)


def kernel(user_idx, item_idx, Iu, user_embedding, item_embedding, user_bias,
           item_bias, yj, W, b):
  iu_pad = jnp.concatenate([Iu, jnp.zeros((8,), jnp.int32)])
  out = _svdpp_sc(user_idx, item_idx, iu_pad,
                  user_embedding.T, item_embedding.T, user_bias, item_bias,
                  yj.T, W, b,
                  user_embedding[TAIL_START:].reshape(-1),
                  item_embedding[TAIL_START:].reshape(-1),
                  yj[TAIL_START:].reshape(-1))
  return out[:1].reshape(1, 1)


# speedup vs baseline: 50.7599x; 1.1100x over previous
import functools
import math

import jax
import jax.numpy as jnp
from jax import lax
from jax.experimental import pallas as pl
from jax.experimental.pallas import tpu as pltpu, tpu_sc as plsc

MU_CONST = 3.5
HLEN = 200
D = 32
L = 16
N_TABLE = 1000000
TAIL_START = (N_TABLE // 128) * 128
LAST_BLOCK = TAIL_START - 128
N_TAIL = N_TABLE - TAIL_START
N_YJW = 13


def _col_lookup(tblT_hbm, blk_slot, idx, sem):
  blkoff = pl.multiple_of(lax.shift_left(lax.shift_right_logical(idx, 7), 7), 128)
  h = pltpu.async_copy(tblT_hbm.at[:, pl.ds(blkoff, 128)], blk_slot, sem)
  col = jnp.bitwise_and(idx, 127)
  return h, col


def _extract(blk_ref, slot, col, iota):
  colv = jnp.full((L,), col, jnp.int32)
  slotv = jnp.full((L,), slot, jnp.int32)
  lo = plsc.load_gather(blk_ref, [slotv, iota, colv])
  hi = plsc.load_gather(blk_ref, [slotv, iota + 16, colv])
  return lo, hi


def _sc_body(uidx_hbm, iidx_hbm, iu_hbm, uembT_hbm, iembT_hbm, ub_hbm, ib_hbm,
             yjT_hbm, w_hbm, b_hbm, out_hbm,
             iu_v, q_v, blk_v, bias_v, part_v, w_v, b_v,
             shared, gath_v, res_v, sem, sem_idx, sem_w, sem_bias):
  cid = lax.axis_index("c")
  sid = lax.axis_index("s")

  @pl.when(cid == 0)
  def _():
    iota = lax.iota(jnp.int32, L)
    zf = jnp.zeros((L,), jnp.float32)

    @pl.when(sid < N_YJW)
    def _():
      base = pl.multiple_of(sid * L, 16)
      c0 = pltpu.async_copy(iu_hbm, iu_v, sem_idx)
      c0.wait()
      gidx = jnp.minimum(jnp.full((L,), base, jnp.int32) + iota, HLEN - 1)
      vec = plsc.load_gather(iu_v, [gidx])
      handles, cols = [], []
      for k in range(L):
        h, col = _col_lookup(yjT_hbm, blk_v.at[k], vec[k], sem)
        handles.append(h)
        cols.append(col)
      for h in handles:
        h.wait()
      acc0, acc1 = zf, zf
      for k in range(L):
        lo, hi = _extract(blk_v, k, cols[k], iota)
        valid = (base + k) < HLEN
        acc0 = acc0 + jnp.where(valid, lo, zf)
        acc1 = acc1 + jnp.where(valid, hi, zf)
      part_v[0, pl.ds(L, L)] = acc0
      part_v[0, pl.ds(2 * L, L)] = acc1
      pltpu.sync_copy(part_v, shared.at[pl.ds(sid, 1)])

    def row_work(idx_hbm, tblT_hbm, bias_hbm, row_slot, bias_slot):
      c0 = pltpu.async_copy(idx_hbm, q_v.at[pl.ds(0, 1)], sem_idx)
      c0.wait()
      u = q_v[...][0]
      h, col = _col_lookup(tblT_hbm, blk_v.at[0], u, sem)
      boff = pl.multiple_of(lax.shift_left(lax.shift_right_logical(u, 4), 4), 16)
      hb = pltpu.async_copy(bias_hbm.at[pl.ds(boff, L)], bias_v, sem_bias)
      h.wait(); hb.wait()
      lo, hi = _extract(blk_v, 0, col, iota)
      part_v[0, pl.ds(L, L)] = lo
      part_v[0, pl.ds(2 * L, L)] = hi
      pltpu.sync_copy(part_v, shared.at[pl.ds(row_slot, 1)])
      lane = jnp.bitwise_and(u, 15)
      sel = jnp.where(iota == lane, bias_v[...], zf)
      part_v[0, pl.ds(L, L)] = sel
      part_v[0, pl.ds(2 * L, L)] = zf
      pltpu.sync_copy(part_v, shared.at[pl.ds(bias_slot, 1)])

    @pl.when(sid == 13)
    def _():
      row_work(uidx_hbm, uembT_hbm, ub_hbm, 13, 15)

    @pl.when(sid == 14)
    def _():
      row_work(iidx_hbm, iembT_hbm, ib_hbm, 14, 16)

    plsc.subcore_barrier()

    @pl.when(sid == 0)
    def _():
      cw = pltpu.async_copy(w_hbm, w_v, sem_w)
      cb = pltpu.async_copy(b_hbm, b_v.at[pl.ds(0, 1)], sem_bias)
      pltpu.sync_copy(shared, gath_v)
      s0, s1 = zf, zf
      for r in range(N_YJW):
        s0 = s0 + gath_v[r, pl.ds(L, L)]
        s1 = s1 + gath_v[r, pl.ds(2 * L, L)]
      cw.wait(); cb.wait()
      inv = jnp.float32(1.0 / math.sqrt(HLEN))
      uv0 = gath_v[13, pl.ds(L, L)] + s0 * inv
      uv1 = gath_v[13, pl.ds(2 * L, L)] + s1 * inv
      p = uv0 * gath_v[14, pl.ds(L, L)] * w_v[0, pl.ds(0, L)] \
          + uv1 * gath_v[14, pl.ds(2 * L, L)] * w_v[0, pl.ds(L, L)]
      dot = jnp.sum(p)
      ubias = jnp.sum(gath_v[15, pl.ds(L, L)])
      ibias = jnp.sum(gath_v[16, pl.ds(L, L)])
      bval = b_v[...][0]
      rating = dot + bval + jnp.float32(MU_CONST) + ubias + ibias
      res_v[...] = jnp.full((L,), rating, jnp.float32)
      pltpu.sync_copy(res_v, out_hbm)


@functools.partial(
    pl.kernel,
    out_type=jax.ShapeDtypeStruct((L,), jnp.float32),
    mesh=plsc.VectorSubcoreMesh(core_axis_name="c", subcore_axis_name="s"),
    compiler_params=pltpu.CompilerParams(use_tc_tiling_on_sc=True,
                                         needs_layout_passes=False),
    scratch_types=[
        pltpu.VMEM((HLEN,), jnp.int32),
        pltpu.VMEM((L,), jnp.int32),
        pltpu.VMEM((L, D, 128), jnp.float32),
        pltpu.VMEM((L,), jnp.float32),
        pltpu.VMEM((1, 2 * D), jnp.float32),
        pltpu.VMEM((1, D), jnp.float32),
        pltpu.VMEM((L,), jnp.float32),
        pltpu.VMEM_SHARED((17, 2 * D), jnp.float32),
        pltpu.VMEM((17, 2 * D), jnp.float32),
        pltpu.VMEM((L,), jnp.float32),
        pltpu.SemaphoreType.DMA,
        pltpu.SemaphoreType.DMA,
        pltpu.SemaphoreType.DMA,
        pltpu.SemaphoreType.DMA,
    ],
)
def _svdpp_sc(*refs):
  _sc_body(*refs)


def kernel(user_idx, item_idx, Iu, user_embedding, item_embedding, user_bias,
           item_bias, yj, W, b):
  out = _svdpp_sc(user_idx, item_idx, Iu,
                  user_embedding.T, item_embedding.T, user_bias, item_bias,
                  yj.T, W, b)
  return out[:1].reshape(1, 1)


# speedup vs baseline: 53.9301x; 1.0625x over previous
import functools
import math

import jax
import jax.numpy as jnp
from jax import lax
from jax.experimental import pallas as pl
from jax.experimental.pallas import tpu as pltpu, tpu_sc as plsc

MU_CONST = 3.5
HLEN = 200
D = 32
L = 16
N_TABLE = 1000000
TAIL_START = (N_TABLE // 128) * 128
LAST_BLOCK = TAIL_START - 128
N_TAIL = N_TABLE - TAIL_START
N_YJW = 13


def _col_lookup(tblT_hbm, blk_slot, idx, sem):
  blkoff = pl.multiple_of(lax.shift_left(lax.shift_right_logical(idx, 7), 7), 128)
  h = pltpu.async_copy(tblT_hbm.at[:, pl.ds(blkoff, 128)], blk_slot, sem)
  col = jnp.bitwise_and(idx, 127)
  return h, col


def _extract(blk_ref, slot, col, iota):
  colv = jnp.full((L,), col, jnp.int32)
  slotv = jnp.full((L,), slot, jnp.int32)
  lo = plsc.load_gather(blk_ref, [slotv, iota, colv])
  hi = plsc.load_gather(blk_ref, [slotv, iota + 16, colv])
  return lo, hi


def _sc_body(uidx_hbm, iidx_hbm, iu_hbm, uembT_hbm, iembT_hbm, ub_hbm, ib_hbm,
             yjT_hbm, w_hbm, b_hbm, out_hbm,
             iu_v, q_v, blk_v, bias_v, part_v, w_v, b_v,
             shared, gath_v, res_v, sem, sem_idx, sem_w, sem_bias):
  cid = lax.axis_index("c")
  sid = lax.axis_index("s")

  @pl.when(cid == 0)
  def _():
    iota = lax.iota(jnp.int32, L)
    zf = jnp.zeros((L,), jnp.float32)

    @pl.when(sid < N_YJW)
    def _():
      base = pl.multiple_of(sid * L, 16)
      c0 = pltpu.async_copy(iu_hbm, iu_v, sem_idx)
      c0.wait()
      gidx = jnp.minimum(jnp.full((L,), base, jnp.int32) + iota, HLEN - 1)
      vec = plsc.load_gather(iu_v, [gidx])
      handles, cols = [], []
      for k in range(L):
        h, col = _col_lookup(yjT_hbm, blk_v.at[k], vec[k], sem)
        handles.append(h)
        cols.append(col)
      for h in handles:
        h.wait()
      acc0, acc1 = zf, zf
      for k in range(L):
        lo, hi = _extract(blk_v, k, cols[k], iota)
        valid = (base + k) < HLEN
        acc0 = acc0 + jnp.where(valid, lo, zf)
        acc1 = acc1 + jnp.where(valid, hi, zf)
      part_v[0, pl.ds(L, L)] = acc0
      part_v[0, pl.ds(2 * L, L)] = acc1
      pltpu.sync_copy(part_v, shared.at[pl.ds(sid, 1)])

    def row_work(idx_hbm, tblT_hbm, bias_hbm, row_slot, bias_slot):
      c0 = pltpu.async_copy(idx_hbm, q_v.at[pl.ds(0, 1)], sem_idx)
      c0.wait()
      u = q_v[...][0]
      h, col = _col_lookup(tblT_hbm, blk_v.at[0], u, sem)
      boff = pl.multiple_of(lax.shift_left(lax.shift_right_logical(u, 4), 4), 16)
      hb = pltpu.async_copy(bias_hbm.at[pl.ds(boff, L)], bias_v, sem_bias)
      h.wait(); hb.wait()
      lo, hi = _extract(blk_v, 0, col, iota)
      part_v[0, pl.ds(L, L)] = lo
      part_v[0, pl.ds(2 * L, L)] = hi
      pltpu.sync_copy(part_v, shared.at[pl.ds(row_slot, 1)])
      lane = jnp.bitwise_and(u, 15)
      sel = jnp.where(iota == lane, bias_v[...], zf)
      part_v[0, pl.ds(L, L)] = sel
      part_v[0, pl.ds(2 * L, L)] = zf
      pltpu.sync_copy(part_v, shared.at[pl.ds(bias_slot, 1)])

    @pl.when(sid == 13)
    def _():
      row_work(uidx_hbm, uembT_hbm, ub_hbm, 13, 15)

    @pl.when(sid == 14)
    def _():
      row_work(iidx_hbm, iembT_hbm, ib_hbm, 14, 16)

    plsc.subcore_barrier()

    @pl.when(sid == 0)
    def _():
      cw = pltpu.async_copy(w_hbm, w_v, sem_w)
      cb = pltpu.async_copy(b_hbm, b_v.at[pl.ds(0, 1)], sem_bias)
      pltpu.sync_copy(shared, gath_v)
      s0, s1 = zf, zf
      for r in range(N_YJW):
        s0 = s0 + gath_v[r, pl.ds(L, L)]
        s1 = s1 + gath_v[r, pl.ds(2 * L, L)]
      cw.wait(); cb.wait()
      inv = jnp.float32(1.0 / math.sqrt(HLEN))
      uv0 = gath_v[13, pl.ds(L, L)] + s0 * inv
      uv1 = gath_v[13, pl.ds(2 * L, L)] + s1 * inv
      p = uv0 * gath_v[14, pl.ds(L, L)] * w_v[0, pl.ds(0, L)] \
          + uv1 * gath_v[14, pl.ds(2 * L, L)] * w_v[0, pl.ds(L, L)]
      dot = jnp.sum(p)
      ubias = jnp.sum(gath_v[15, pl.ds(L, L)])
      ibias = jnp.sum(gath_v[16, pl.ds(L, L)])
      bval = b_v[...][0]
      rating = dot + bval + jnp.float32(MU_CONST) + ubias + ibias
      res_v[...] = jnp.full((L,), rating, jnp.float32)
      pltpu.sync_copy(res_v, out_hbm)


@functools.partial(
    pl.kernel,
    out_type=jax.ShapeDtypeStruct((L,), jnp.float32),
    mesh=plsc.VectorSubcoreMesh(core_axis_name="c", subcore_axis_name="s", num_cores=1),
    compiler_params=pltpu.CompilerParams(use_tc_tiling_on_sc=True,
                                         needs_layout_passes=False),
    scratch_types=[
        pltpu.VMEM((HLEN,), jnp.int32),
        pltpu.VMEM((L,), jnp.int32),
        pltpu.VMEM((L, D, 128), jnp.float32),
        pltpu.VMEM((L,), jnp.float32),
        pltpu.VMEM((1, 2 * D), jnp.float32),
        pltpu.VMEM((1, D), jnp.float32),
        pltpu.VMEM((L,), jnp.float32),
        pltpu.VMEM_SHARED((17, 2 * D), jnp.float32),
        pltpu.VMEM((17, 2 * D), jnp.float32),
        pltpu.VMEM((L,), jnp.float32),
        pltpu.SemaphoreType.DMA,
        pltpu.SemaphoreType.DMA,
        pltpu.SemaphoreType.DMA,
        pltpu.SemaphoreType.DMA,
    ],
)
def _svdpp_sc(*refs):
  _sc_body(*refs)


def kernel(user_idx, item_idx, Iu, user_embedding, item_embedding, user_bias,
           item_bias, yj, W, b):
  out = _svdpp_sc(user_idx, item_idx, Iu,
                  user_embedding.T, item_embedding.T, user_bias, item_bias,
                  yj.T, W, b)
  return out[:1].reshape(1, 1)


# speedup vs baseline: 57.7285x; 1.0704x over previous
import functools
import math

import jax
import jax.numpy as jnp
from jax import lax
from jax.experimental import pallas as pl
from jax.experimental.pallas import tpu as pltpu, tpu_sc as plsc

MU_CONST = 3.5
HLEN = 200
D = 32
L = 16
N_TABLE = 1000000
TAIL_START = (N_TABLE // 128) * 128
LAST_BLOCK = TAIL_START - 128
N_TAIL = N_TABLE - TAIL_START
N_YJW = 16
PER_W = 13


def _col_lookup(tblT_hbm, blk_slot, idx, sem):
  blkoff = pl.multiple_of(lax.shift_left(lax.shift_right_logical(idx, 7), 7), 128)
  h = pltpu.async_copy(tblT_hbm.at[:, pl.ds(blkoff, 128)], blk_slot, sem)
  col = jnp.bitwise_and(idx, 127)
  return h, col


def _extract(blk_ref, slot, col, iota):
  colv = jnp.full((L,), col, jnp.int32)
  slotv = jnp.full((L,), slot, jnp.int32)
  lo = plsc.load_gather(blk_ref, [slotv, iota, colv])
  hi = plsc.load_gather(blk_ref, [slotv, iota + 16, colv])
  return lo, hi


def _sc_body(uidx_hbm, iidx_hbm, iu_hbm, uembT_hbm, iembT_hbm, ub_hbm, ib_hbm,
             yjT_hbm, w_hbm, b_hbm, out_hbm,
             iu_v, q_v, blk_v, bias_v, part_v, w_v, b_v,
             shared, gath_v, res_v, sem, sem_idx, sem_w, sem_bias):
  cid = lax.axis_index("c")
  sid = lax.axis_index("s")

  @pl.when(cid == 0)
  def _():
    iota = lax.iota(jnp.int32, L)
    zf = jnp.zeros((L,), jnp.float32)

    @pl.when(sid < N_YJW)
    def _():
      base = sid * PER_W
      c0 = pltpu.async_copy(iu_hbm, iu_v, sem_idx)
      c0.wait()
      gidx = jnp.minimum(jnp.full((L,), base, jnp.int32) + iota, HLEN - 1)
      vec = plsc.load_gather(iu_v, [gidx])
      handles, cols = [], []
      for k in range(PER_W):
        h, col = _col_lookup(yjT_hbm, blk_v.at[k], vec[k], sem)
        handles.append(h)
        cols.append(col)
      for h in handles:
        h.wait()
      acc0, acc1 = zf, zf
      for k in range(PER_W):
        lo, hi = _extract(blk_v, k, cols[k], iota)
        valid = (base + k) < HLEN
        acc0 = acc0 + jnp.where(valid, lo, zf)
        acc1 = acc1 + jnp.where(valid, hi, zf)
      part_v[0, pl.ds(L, L)] = acc0
      part_v[0, pl.ds(2 * L, L)] = acc1
      pltpu.sync_copy(part_v, shared.at[pl.ds(sid, 1)])

    def row_work(idx_hbm, tblT_hbm, bias_hbm, row_slot, bias_slot):
      c0 = pltpu.async_copy(idx_hbm, q_v.at[pl.ds(0, 1)], sem_idx)
      c0.wait()
      u = q_v[...][0]
      h, col = _col_lookup(tblT_hbm, blk_v.at[13], u, sem)
      boff = pl.multiple_of(lax.shift_left(lax.shift_right_logical(u, 4), 4), 16)
      hb = pltpu.async_copy(bias_hbm.at[pl.ds(boff, L)], bias_v, sem_bias)
      h.wait(); hb.wait()
      lo, hi = _extract(blk_v, 13, col, iota)
      part_v[0, pl.ds(L, L)] = lo
      part_v[0, pl.ds(2 * L, L)] = hi
      pltpu.sync_copy(part_v, shared.at[pl.ds(row_slot, 1)])
      lane = jnp.bitwise_and(u, 15)
      sel = jnp.where(iota == lane, bias_v[...], zf)
      part_v[0, pl.ds(L, L)] = sel
      part_v[0, pl.ds(2 * L, L)] = zf
      pltpu.sync_copy(part_v, shared.at[pl.ds(bias_slot, 1)])

    @pl.when(sid == 14)
    def _():
      row_work(uidx_hbm, uembT_hbm, ub_hbm, 17, 19)

    @pl.when(sid == 15)
    def _():
      row_work(iidx_hbm, iembT_hbm, ib_hbm, 18, 20)

    plsc.subcore_barrier()

    @pl.when(sid == 0)
    def _():
      cw = pltpu.async_copy(w_hbm, w_v, sem_w)
      cb = pltpu.async_copy(b_hbm, b_v.at[pl.ds(0, 1)], sem_bias)
      pltpu.sync_copy(shared, gath_v)
      s0, s1 = zf, zf
      for r in range(N_YJW):
        s0 = s0 + gath_v[r, pl.ds(L, L)]
        s1 = s1 + gath_v[r, pl.ds(2 * L, L)]
      cw.wait(); cb.wait()
      inv = jnp.float32(1.0 / math.sqrt(HLEN))
      uv0 = gath_v[17, pl.ds(L, L)] + s0 * inv
      uv1 = gath_v[17, pl.ds(2 * L, L)] + s1 * inv
      p = uv0 * gath_v[18, pl.ds(L, L)] * w_v[0, pl.ds(0, L)] \
          + uv1 * gath_v[18, pl.ds(2 * L, L)] * w_v[0, pl.ds(L, L)]
      dot = jnp.sum(p)
      ubias = jnp.sum(gath_v[19, pl.ds(L, L)])
      ibias = jnp.sum(gath_v[20, pl.ds(L, L)])
      bval = b_v[...][0]
      rating = dot + bval + jnp.float32(MU_CONST) + ubias + ibias
      res_v[...] = jnp.full((L,), rating, jnp.float32)
      pltpu.sync_copy(res_v, out_hbm)


@functools.partial(
    pl.kernel,
    out_type=jax.ShapeDtypeStruct((L,), jnp.float32),
    mesh=plsc.VectorSubcoreMesh(core_axis_name="c", subcore_axis_name="s", num_cores=1),
    compiler_params=pltpu.CompilerParams(use_tc_tiling_on_sc=True,
                                         needs_layout_passes=False),
    scratch_types=[
        pltpu.VMEM((HLEN,), jnp.int32),
        pltpu.VMEM((L,), jnp.int32),
        pltpu.VMEM((14, D, 128), jnp.float32),
        pltpu.VMEM((L,), jnp.float32),
        pltpu.VMEM((1, 2 * D), jnp.float32),
        pltpu.VMEM((1, D), jnp.float32),
        pltpu.VMEM((L,), jnp.float32),
        pltpu.VMEM_SHARED((21, 2 * D), jnp.float32),
        pltpu.VMEM((21, 2 * D), jnp.float32),
        pltpu.VMEM((L,), jnp.float32),
        pltpu.SemaphoreType.DMA,
        pltpu.SemaphoreType.DMA,
        pltpu.SemaphoreType.DMA,
        pltpu.SemaphoreType.DMA,
    ],
)
def _svdpp_sc(*refs):
  _sc_body(*refs)


def kernel(user_idx, item_idx, Iu, user_embedding, item_embedding, user_bias,
           item_bias, yj, W, b):
  out = _svdpp_sc(user_idx, item_idx, Iu,
                  user_embedding.T, item_embedding.T, user_bias, item_bias,
                  yj.T, W, b)
  return out[:1].reshape(1, 1)
